# Initial kernel scaffold; baseline (speedup 1.0000x reference)
import jax
import jax.numpy as jnp
from jax.experimental import pallas as pl


def kernel(x, edge_index, W1, b1, W2, b2, W3, b3, W4, b4, W5, b5):
    raise NotImplementedError("write your pallas kernel here")


# speedup vs baseline: 8.0812x; 8.0812x over previous
import functools

import jax
import jax.numpy as jnp
from jax import lax
from jax.experimental import pallas as pl
from jax.experimental.pallas import tpu as pltpu
from jax.experimental.pallas import tpu_sc as plsc

_N = 50000
_E = 800000

_NC = 2
_NS = 16
_NW = _NC * _NS

_CHUNK = 128
_NCHUNKS = _E // _CHUNK
_TPW = -(-_NCHUNKS // _NW)

_ZC = 400
_NZ = _N // _ZC
_ZPS = -(-_NZ // _NS)

_R = 2000
_G = _N // _R


def _mesh():
    return plsc.VectorSubcoreMesh(
        core_axis_name="c", subcore_axis_name="s",
        num_cores=_NC, num_subcores=_NS)


def _fill(buf, d, value):
    v16 = jnp.full((16,), value, jnp.float32)
    if d == 1:
        @pl.loop(0, _ZC // 16)
        def _(i):
            buf[pl.ds(i * 16, 16)] = v16
    else:
        @pl.loop(0, _ZC)
        def _(r):
            for c in range(d // 16):
                buf[r, pl.ds(c * 16, 16)] = v16


def _row_chunks(s, body):
    @pl.loop(0, _ZPS)
    def _(t):
        z = s + t * _NS
        @pl.when(z < _NZ)
        def _():
            body(pl.multiple_of(z * _ZC, 8))


def _make_agg(d):
    if d == 1:
        out_t = jax.ShapeDtypeStruct((_N,), jnp.float32)
        acc_t = pltpu.VMEM_SHARED((_N,), jnp.float32)
        rows_t = pltpu.VMEM((_CHUNK,), jnp.float32)
        zbuf_t = pltpu.VMEM((_ZC,), jnp.float32)
    else:
        out_t = jax.ShapeDtypeStruct((_N, d), jnp.float32)
        acc_t = pltpu.VMEM_SHARED((_N, d), jnp.float32)
        rows_t = pltpu.VMEM((_CHUNK, d), jnp.float32)
        zbuf_t = pltpu.VMEM((_ZC, d), jnp.float32)

    @functools.partial(
        pl.kernel,
        out_type=(out_t, out_t),
        mesh=_mesh(),
        scratch_types=[
            acc_t,
            pltpu.VMEM((_CHUNK,), jnp.int32),
            pltpu.VMEM((_CHUNK,), jnp.int32),
            rows_t,
            zbuf_t,
            pltpu.SemaphoreType.DMA,
        ],
        compiler_params=pltpu.CompilerParams(use_tc_tiling_on_sc=False),
    )
    def agg(u_hbm, src_hbm, dst_hbm, out0_hbm, out1_hbm, acc,
            isrc, idst, rows, zbuf, sem):
        c = lax.axis_index("c")
        s = lax.axis_index("s")
        w = s * _NC + c

        _fill(zbuf, d, 0.0)
        _row_chunks(s, lambda off: pltpu.sync_copy(zbuf, acc.at[pl.ds(off, _ZC)]))
        plsc.subcore_barrier()

        @pl.loop(0, _TPW)
        def _(t):
            g = w + t * _NW
            @pl.when(g < _NCHUNKS)
            def _():
                base = pl.multiple_of(g * _CHUNK, 8)
                pltpu.sync_copy(src_hbm.at[pl.ds(base, _CHUNK)], isrc)
                pltpu.sync_copy(dst_hbm.at[pl.ds(base, _CHUNK)], idst)
                pltpu.async_copy(u_hbm.at[isrc], rows, sem).wait()
                pltpu.sync_copy(rows, acc.at[idst], add=True)

        plsc.subcore_barrier()

        def _out_to(out_hbm):
            def _out(off):
                pltpu.sync_copy(acc.at[pl.ds(off, _ZC)], zbuf)
                pltpu.sync_copy(zbuf, out_hbm.at[pl.ds(off, _ZC)])
            return _out

        @pl.when(c == 0)
        def _():
            _row_chunks(s, _out_to(out0_hbm))

        @pl.when(c == 1)
        def _():
            _row_chunks(s, _out_to(out1_hbm))

    return agg


def _make_deg():
    out_t = jax.ShapeDtypeStruct((_N,), jnp.float32)

    @functools.partial(
        pl.kernel,
        out_type=(out_t, out_t, out_t, out_t),
        mesh=_mesh(),
        scratch_types=[
            pltpu.VMEM_SHARED((_N,), jnp.float32),
            pltpu.VMEM_SHARED((_N,), jnp.float32),
            pltpu.VMEM((_CHUNK,), jnp.int32),
            pltpu.VMEM((_CHUNK,), jnp.int32),
            pltpu.VMEM((_CHUNK,), jnp.float32),
            pltpu.VMEM((_ZC,), jnp.float32),
        ],
    )
    def deg(src_hbm, dst_hbm, od0_hbm, od1_hbm, id0_hbm, id1_hbm,
            acco, acci, isrc, idst, ones_v, zbuf):
        c = lax.axis_index("c")
        s = lax.axis_index("s")
        w = s * _NC + c

        _fill(zbuf, 1, 0.0)
        v16 = jnp.full((16,), 1.0, jnp.float32)
        @pl.loop(0, _CHUNK // 16)
        def _(i):
            ones_v[pl.ds(i * 16, 16)] = v16

        def _zero(off):
            pltpu.sync_copy(zbuf, acco.at[pl.ds(off, _ZC)])
            pltpu.sync_copy(zbuf, acci.at[pl.ds(off, _ZC)])
        _row_chunks(s, _zero)
        plsc.subcore_barrier()

        @pl.loop(0, _TPW)
        def _(t):
            g = w + t * _NW
            @pl.when(g < _NCHUNKS)
            def _():
                base = pl.multiple_of(g * _CHUNK, 8)
                pltpu.sync_copy(src_hbm.at[pl.ds(base, _CHUNK)], isrc)
                pltpu.sync_copy(dst_hbm.at[pl.ds(base, _CHUNK)], idst)
                pltpu.sync_copy(ones_v, acco.at[isrc], add=True)
                pltpu.sync_copy(ones_v, acci.at[idst], add=True)

        plsc.subcore_barrier()

        def _out_to(od_hbm, id_hbm):
            def _out(off):
                pltpu.sync_copy(acco.at[pl.ds(off, _ZC)], zbuf)
                pltpu.sync_copy(zbuf, od_hbm.at[pl.ds(off, _ZC)])
                pltpu.sync_copy(acci.at[pl.ds(off, _ZC)], zbuf)
                pltpu.sync_copy(zbuf, id_hbm.at[pl.ds(off, _ZC)])
            return _out

        @pl.when(c == 0)
        def _():
            _row_chunks(s, _out_to(od0_hbm, id0_hbm))

        @pl.when(c == 1)
        def _():
            _row_chunks(s, _out_to(od1_hbm, id1_hbm))

    return deg


def _vspec(d):
    return pl.BlockSpec((_R, d), lambda i: (i, 0))


def _wspec(shape):
    ndim = len(shape)
    return pl.BlockSpec(shape, lambda i, _nd=ndim: (0,) * _nd)


def _f32(*shape):
    return jax.ShapeDtypeStruct(shape, jnp.float32)


def _tc_call(body, in_specs, out_specs, out_shape, args):
    if len(out_specs) == 1:
        out_specs = out_specs[0]
    return pl.pallas_call(
        body,
        grid=(_G,),
        in_specs=in_specs,
        out_specs=out_specs,
        out_shape=out_shape,
    )(*args)


def kernel(x, edge_index, W1, b1, W2, b2, W3, b3, W4, b4, W5, b5):
    src = edge_index[0]
    dst = edge_index[1]
    x2 = x.reshape(_N, 1)
    b1r, b2r, b3r, b4r, b5r = (b.reshape(1, -1) for b in (b1, b2, b3, b4, b5))

    agg1 = _make_agg(1)
    agg16 = _make_agg(16)
    agg32 = _make_agg(32)

    def v2(a):
        return a.reshape(_N, 1)

    od0, od1, id0, id1 = _make_deg()(src, dst)

    def t0(od0_r, od1_r, id0_r, id1_r, x_r, on_o, in_o, u1_o):
        od = od0_r[...] + od1_r[...]
        ideg = id0_r[...] + id1_r[...]
        on_o[...] = jnp.where(od > 0, lax.rsqrt(jnp.maximum(od, 1.0)), 0.0)
        in_o[...] = jnp.where(ideg > 0, lax.rsqrt(jnp.maximum(ideg, 1.0)), 0.0)
        u1_o[...] = on_o[...] * x_r[...]

    on, inn, u1 = _tc_call(
        t0,
        [_vspec(1)] * 5,
        [_vspec(1)] * 3,
        (_f32(_N, 1), _f32(_N, 1), _f32(_N, 1)),
        (v2(od0), v2(od1), v2(id0), v2(id1), x2))

    s10, s11 = agg1(u1.reshape(_N), src, dst)

    def t1(s0_r, s1_r, in_r, on_r, w_r, b_r, u_o):
        sv = in_r[...] * (s0_r[...] + s1_r[...])
        u_o[...] = on_r[...] * jax.nn.relu(sv * w_r[...] + b_r[...])

    u2 = _tc_call(
        t1,
        [_vspec(1)] * 4 + [_wspec((1, 16)), _wspec((1, 16))],
        [_vspec(16)],
        _f32(_N, 16),
        (v2(s10), v2(s11), inn, on, W1, b1r))

    s20, s21 = agg16(u2, src, dst)

    def t2(s0_r, s1_r, in_r, on_r, w_r, b_r, u_o):
        sv = in_r[...] * (s0_r[...] + s1_r[...])
        h = jax.nn.relu(jnp.dot(sv, w_r[...],
                                preferred_element_type=jnp.float32) + b_r[...])
        u_o[...] = on_r[...] * h

    u3 = _tc_call(
        t2,
        [_vspec(16), _vspec(16), _vspec(1), _vspec(1),
         _wspec((16, 32)), _wspec((1, 32))],
        [_vspec(32)],
        _f32(_N, 32),
        (s20, s21, inn, on, W2, b2r))

    s30, s31 = agg32(u3, src, dst)

    def t3(s0_r, s1_r, in_r, on_r, w3_r, b3_r, w4_r, u_o):
        sv = in_r[...] * (s0_r[...] + s1_r[...])
        h = jax.nn.relu(jnp.dot(sv, w3_r[...],
                                preferred_element_type=jnp.float32) + b3_r[...])
        u_o[...] = on_r[...] * jnp.dot(h, w4_r[...],
                                       preferred_element_type=jnp.float32)

    u4 = _tc_call(
        t3,
        [_vspec(32), _vspec(32), _vspec(1), _vspec(1),
         _wspec((32, 64)), _wspec((1, 64)), _wspec((64, 32))],
        [_vspec(32)],
        _f32(_N, 32),
        (s30, s31, inn, on, W3, b3r, W4))

    s40, s41 = agg32(u4, src, dst)

    def t4(s0_r, s1_r, in_r, on_r, b4_r, w5_r, u_o):
        h = jax.nn.relu(in_r[...] * (s0_r[...] + s1_r[...]) + b4_r[...])
        u_o[...] = on_r[...] * jnp.dot(h, w5_r[...],
                                       preferred_element_type=jnp.float32)

    u5 = _tc_call(
        t4,
        [_vspec(32), _vspec(32), _vspec(1), _vspec(1),
         _wspec((1, 32)), _wspec((32, 1))],
        [_vspec(1)],
        _f32(_N, 1),
        (s40, s41, inn, on, b4r, W5))

    s50, s51 = agg1(u5.reshape(_N), src, dst)

    def t5(s0_r, s1_r, in_r, b_r, y_o):
        y_o[...] = jax.nn.sigmoid(
            in_r[...] * (s0_r[...] + s1_r[...]) + b_r[...])

    y = _tc_call(
        t5,
        [_vspec(1)] * 3 + [_wspec((1, 1))],
        [_vspec(1)],
        _f32(_N, 1),
        (v2(s50), v2(s51), inn, b5r))

    return y.reshape(1, _N)


# speedup vs baseline: 16.3763x; 2.0265x over previous
import functools

import jax
import jax.numpy as jnp
from jax import lax
from jax.experimental import pallas as pl
from jax.experimental.pallas import tpu as pltpu
from jax.experimental.pallas import tpu_sc as plsc

_N = 50000
_E = 800000

_NC = 2
_NS = 16
_NW = _NC * _NS

_CHUNKS = {1: 1600, 16: 800, 32: 400}

_ZC = 400
_NZ = _N // _ZC
_ZPS = -(-_NZ // _NS)

_R = 2000
_G = _N // _R


def _mesh():
    return plsc.VectorSubcoreMesh(
        core_axis_name="c", subcore_axis_name="s",
        num_cores=_NC, num_subcores=_NS)


def _fill(buf, d, rows, value):
    v16 = jnp.full((16,), value, jnp.float32)
    if d == 1:
        @pl.loop(0, rows // 16)
        def _(i):
            buf[pl.ds(i * 16, 16)] = v16
    else:
        @pl.loop(0, rows)
        def _(r):
            for c in range(d // 16):
                buf[r, pl.ds(c * 16, 16)] = v16


def _row_chunks(s, body):
    @pl.loop(0, _ZPS)
    def _(t):
        z = s + t * _NS
        @pl.when(z < _NZ)
        def _():
            body(pl.multiple_of(z * _ZC, 8))


def _make_agg(d):
    chunk = _CHUNKS[d]
    nchunks = _E // chunk
    tpw = -(-nchunks // _NW)
    if d == 1:
        out_t = jax.ShapeDtypeStruct((_N,), jnp.float32)
        acc_t = pltpu.VMEM_SHARED((_N,), jnp.float32)
        rows_t = pltpu.VMEM((chunk,), jnp.float32)
    else:
        out_t = jax.ShapeDtypeStruct((_N, d), jnp.float32)
        acc_t = pltpu.VMEM_SHARED((_N, d), jnp.float32)
        rows_t = pltpu.VMEM((chunk, d), jnp.float32)

    @functools.partial(
        pl.kernel,
        out_type=(out_t, out_t),
        mesh=_mesh(),
        scratch_types=[
            acc_t,
            pltpu.VMEM((chunk,), jnp.int32),
            pltpu.VMEM((chunk,), jnp.int32),
            rows_t,
            pltpu.SemaphoreType.DMA,
        ],
        compiler_params=pltpu.CompilerParams(use_tc_tiling_on_sc=False),
    )
    def agg(u_hbm, src_hbm, dst_hbm, out0_hbm, out1_hbm, acc,
            isrc, idst, rows, sem):
        c = lax.axis_index("c")
        s = lax.axis_index("s")
        w = s * _NC + c
        zview = rows.at[pl.ds(0, _ZC)]

        _fill(rows, d, chunk, 0.0)
        _row_chunks(s, lambda off: pltpu.sync_copy(zview, acc.at[pl.ds(off, _ZC)]))
        plsc.subcore_barrier()

        @pl.loop(0, tpw)
        def _(t):
            g = w + t * _NW
            @pl.when(g < nchunks)
            def _():
                base = pl.multiple_of(g * chunk, 8)
                pltpu.sync_copy(src_hbm.at[pl.ds(base, chunk)], isrc)
                pltpu.sync_copy(dst_hbm.at[pl.ds(base, chunk)], idst)
                pltpu.async_copy(u_hbm.at[isrc], rows, sem).wait()
                pltpu.sync_copy(rows, acc.at[idst], add=True)

        plsc.subcore_barrier()

        def _out_to(out_hbm):
            def _out(off):
                pltpu.sync_copy(acc.at[pl.ds(off, _ZC)], zview)
                pltpu.sync_copy(zview, out_hbm.at[pl.ds(off, _ZC)])
            return _out

        @pl.when(c == 0)
        def _():
            _row_chunks(s, _out_to(out0_hbm))

        @pl.when(c == 1)
        def _():
            _row_chunks(s, _out_to(out1_hbm))

    return agg


def _make_deg():
    out_t = jax.ShapeDtypeStruct((_N,), jnp.float32)
    chunk = _CHUNKS[1]
    nchunks = _E // chunk
    tpw = -(-nchunks // _NW)

    @functools.partial(
        pl.kernel,
        out_type=(out_t, out_t, out_t, out_t),
        mesh=_mesh(),
        scratch_types=[
            pltpu.VMEM_SHARED((_N,), jnp.float32),
            pltpu.VMEM_SHARED((_N,), jnp.float32),
            pltpu.VMEM((chunk,), jnp.int32),
            pltpu.VMEM((chunk,), jnp.int32),
            pltpu.VMEM((chunk,), jnp.float32),
            pltpu.VMEM((_ZC,), jnp.float32),
        ],
    )
    def deg(src_hbm, dst_hbm, od0_hbm, od1_hbm, id0_hbm, id1_hbm,
            acco, acci, isrc, idst, ones_v, zbuf):
        c = lax.axis_index("c")
        s = lax.axis_index("s")
        w = s * _NC + c

        _fill(zbuf, 1, _ZC, 0.0)
        _fill(ones_v, 1, chunk, 1.0)

        def _zero(off):
            pltpu.sync_copy(zbuf, acco.at[pl.ds(off, _ZC)])
            pltpu.sync_copy(zbuf, acci.at[pl.ds(off, _ZC)])
        _row_chunks(s, _zero)
        plsc.subcore_barrier()

        @pl.loop(0, tpw)
        def _(t):
            g = w + t * _NW
            @pl.when(g < nchunks)
            def _():
                base = pl.multiple_of(g * chunk, 8)
                pltpu.sync_copy(src_hbm.at[pl.ds(base, chunk)], isrc)
                pltpu.sync_copy(dst_hbm.at[pl.ds(base, chunk)], idst)
                pltpu.sync_copy(ones_v, acco.at[isrc], add=True)
                pltpu.sync_copy(ones_v, acci.at[idst], add=True)

        plsc.subcore_barrier()

        def _out_to(od_hbm, id_hbm):
            def _out(off):
                pltpu.sync_copy(acco.at[pl.ds(off, _ZC)], zbuf)
                pltpu.sync_copy(zbuf, od_hbm.at[pl.ds(off, _ZC)])
                pltpu.sync_copy(acci.at[pl.ds(off, _ZC)], zbuf)
                pltpu.sync_copy(zbuf, id_hbm.at[pl.ds(off, _ZC)])
            return _out

        @pl.when(c == 0)
        def _():
            _row_chunks(s, _out_to(od0_hbm, id0_hbm))

        @pl.when(c == 1)
        def _():
            _row_chunks(s, _out_to(od1_hbm, id1_hbm))

    return deg


def _vspec(d):
    return pl.BlockSpec((_R, d), lambda i: (i, 0))


def _wspec(shape):
    ndim = len(shape)
    return pl.BlockSpec(shape, lambda i, _nd=ndim: (0,) * _nd)


def _f32(*shape):
    return jax.ShapeDtypeStruct(shape, jnp.float32)


def _tc_call(body, in_specs, out_specs, out_shape, args):
    if len(out_specs) == 1:
        out_specs = out_specs[0]
    return pl.pallas_call(
        body,
        grid=(_G,),
        in_specs=in_specs,
        out_specs=out_specs,
        out_shape=out_shape,
    )(*args)


def kernel(x, edge_index, W1, b1, W2, b2, W3, b3, W4, b4, W5, b5):
    src = edge_index[0]
    dst = edge_index[1]
    x2 = x.reshape(_N, 1)
    b1r, b2r, b3r, b4r, b5r = (b.reshape(1, -1) for b in (b1, b2, b3, b4, b5))

    agg1 = _make_agg(1)
    agg16 = _make_agg(16)
    agg32 = _make_agg(32)

    def v2(a):
        return a.reshape(_N, 1)

    od0, od1, id0, id1 = _make_deg()(src, dst)

    def t0(od0_r, od1_r, id0_r, id1_r, x_r, on_o, in_o, u1_o):
        od = od0_r[...] + od1_r[...]
        ideg = id0_r[...] + id1_r[...]
        on_o[...] = jnp.where(od > 0, lax.rsqrt(jnp.maximum(od, 1.0)), 0.0)
        in_o[...] = jnp.where(ideg > 0, lax.rsqrt(jnp.maximum(ideg, 1.0)), 0.0)
        u1_o[...] = on_o[...] * x_r[...]

    on, inn, u1 = _tc_call(
        t0,
        [_vspec(1)] * 5,
        [_vspec(1)] * 3,
        (_f32(_N, 1), _f32(_N, 1), _f32(_N, 1)),
        (v2(od0), v2(od1), v2(id0), v2(id1), x2))

    s10, s11 = agg1(u1.reshape(_N), src, dst)

    def t1(s0_r, s1_r, in_r, on_r, w_r, b_r, u_o):
        sv = in_r[...] * (s0_r[...] + s1_r[...])
        u_o[...] = on_r[...] * jax.nn.relu(sv * w_r[...] + b_r[...])

    u2 = _tc_call(
        t1,
        [_vspec(1)] * 4 + [_wspec((1, 16)), _wspec((1, 16))],
        [_vspec(16)],
        _f32(_N, 16),
        (v2(s10), v2(s11), inn, on, W1, b1r))

    s20, s21 = agg16(u2, src, dst)

    def t2(s0_r, s1_r, in_r, on_r, w_r, b_r, u_o):
        sv = in_r[...] * (s0_r[...] + s1_r[...])
        h = jax.nn.relu(jnp.dot(sv, w_r[...],
                                preferred_element_type=jnp.float32) + b_r[...])
        u_o[...] = on_r[...] * h

    u3 = _tc_call(
        t2,
        [_vspec(16), _vspec(16), _vspec(1), _vspec(1),
         _wspec((16, 32)), _wspec((1, 32))],
        [_vspec(32)],
        _f32(_N, 32),
        (s20, s21, inn, on, W2, b2r))

    s30, s31 = agg32(u3, src, dst)

    def t3(s0_r, s1_r, in_r, on_r, w3_r, b3_r, w4_r, u_o):
        sv = in_r[...] * (s0_r[...] + s1_r[...])
        h = jax.nn.relu(jnp.dot(sv, w3_r[...],
                                preferred_element_type=jnp.float32) + b3_r[...])
        u_o[...] = on_r[...] * jnp.dot(h, w4_r[...],
                                       preferred_element_type=jnp.float32)

    u4 = _tc_call(
        t3,
        [_vspec(32), _vspec(32), _vspec(1), _vspec(1),
         _wspec((32, 64)), _wspec((1, 64)), _wspec((64, 32))],
        [_vspec(32)],
        _f32(_N, 32),
        (s30, s31, inn, on, W3, b3r, W4))

    s40, s41 = agg32(u4, src, dst)

    def t4(s0_r, s1_r, in_r, on_r, b4_r, w5_r, u_o):
        h = jax.nn.relu(in_r[...] * (s0_r[...] + s1_r[...]) + b4_r[...])
        u_o[...] = on_r[...] * jnp.dot(h, w5_r[...],
                                       preferred_element_type=jnp.float32)

    u5 = _tc_call(
        t4,
        [_vspec(32), _vspec(32), _vspec(1), _vspec(1),
         _wspec((1, 32)), _wspec((32, 1))],
        [_vspec(1)],
        _f32(_N, 1),
        (s40, s41, inn, on, b4r, W5))

    s50, s51 = agg1(u5.reshape(_N), src, dst)

    def t5(s0_r, s1_r, in_r, b_r, y_o):
        y_o[...] = jax.nn.sigmoid(
            in_r[...] * (s0_r[...] + s1_r[...]) + b_r[...])

    y = _tc_call(
        t5,
        [_vspec(1)] * 3 + [_wspec((1, 1))],
        [_vspec(1)],
        _f32(_N, 1),
        (v2(s50), v2(s51), inn, b5r))

    return y.reshape(1, _N)
